# Initial kernel scaffold; baseline (speedup 1.0000x reference)
import jax
import jax.numpy as jnp
from jax.experimental import pallas as pl


def kernel(z, pos_edge_index, neg_edge_index):
    raise NotImplementedError("write your pallas kernel here")


# speedup vs baseline: 2.3491x; 2.3491x over previous
import functools

import jax
import jax.numpy as jnp
from jax import lax
from jax.experimental import pallas as pl
from jax.experimental.pallas import tpu as pltpu
from jax.experimental.pallas import tpu_sc as plsc

D = 128
E = 640000
NW = 32
PER_W = E // NW
C = 80
NCHUNK = PER_W // C
L = 16


def _sc_scores(z, src, dst):
  mesh = plsc.VectorSubcoreMesh(core_axis_name="c", subcore_axis_name="s")

  @functools.partial(
      pl.kernel,
      out_type=jax.ShapeDtypeStruct((E,), jnp.float32),
      mesh=mesh,
      compiler_params=pltpu.CompilerParams(needs_layout_passes=False),
      scratch_types=[
          pltpu.VMEM((C,), jnp.int32),
          pltpu.VMEM((C,), jnp.int32),
          pltpu.VMEM((C, D), jnp.float32),
          pltpu.VMEM((C, D), jnp.float32),
          pltpu.VMEM((L * L,), jnp.float32),
          pltpu.VMEM((C,), jnp.float32),
          pltpu.SemaphoreType.DMA,
          pltpu.SemaphoreType.DMA,
      ],
  )
  def k(z_hbm, src_hbm, dst_hbm, out_hbm,
        sidx, didx, srows, drows, accbuf, sv, sem1, sem2):
    wid = lax.axis_index("s") * 2 + lax.axis_index("c")
    base0 = wid * PER_W
    lanes = lax.iota(jnp.int32, L)

    @pl.loop(0, NCHUNK)
    def _(i):
      base = base0 + i * C
      pltpu.sync_copy(src_hbm.at[pl.ds(base, C)], sidx)
      pltpu.sync_copy(dst_hbm.at[pl.ds(base, C)], didx)
      cp1 = pltpu.async_copy(z_hbm.at[sidx], srows, sem1)
      cp2 = pltpu.async_copy(z_hbm.at[didx], drows, sem2)
      cp1.wait()
      cp2.wait()
      for g in range(C // L):
        for e in range(L):
          r = g * L + e
          acc = srows[r, pl.ds(0, L)] * drows[r, pl.ds(0, L)]
          for kk in range(1, D // L):
            acc = acc + srows[r, pl.ds(kk * L, L)] * drows[r, pl.ds(kk * L, L)]
          accbuf[pl.ds(e * L, L)] = acc
        scv = plsc.load_gather(accbuf, [lanes * L])
        for j in range(1, L):
          scv = scv + plsc.load_gather(accbuf, [lanes * L + j])
        sv[pl.ds(g * L, L)] = scv
      pltpu.sync_copy(sv, out_hbm.at[pl.ds(base, C)])

  return k(z, src, dst)


def _bce_mean_tc(scores2d):
  rows_total = E // 128

  def body(s_ref, o_ref):
    s = s_ref[...]
    row = lax.broadcasted_iota(jnp.int32, (rows_total, 128), 0)
    label = jnp.where(row < rows_total // 2, 1.0, 0.0).astype(jnp.float32)
    loss = jnp.maximum(s, 0.0) - s * label + jnp.log1p(jnp.exp(-jnp.abs(s)))
    o_ref[...] = (jnp.sum(loss) / E).reshape(1, 1)

  return pl.pallas_call(
      body,
      out_shape=jax.ShapeDtypeStruct((1, 1), jnp.float32),
  )(scores2d)


@jax.jit
def kernel(z, pos_edge_index, neg_edge_index):
  src = jnp.concatenate(
      [pos_edge_index[0], neg_edge_index[0]]).astype(jnp.int32)
  dst = jnp.concatenate(
      [pos_edge_index[1], neg_edge_index[1]]).astype(jnp.int32)
  scores = _sc_scores(z, src, dst)
  loss = _bce_mean_tc(scores.reshape(E // 128, 128))
  return loss[0, 0]
